# baseline (speedup 1.0000x reference)
import jax
import jax.numpy as jnp
from jax.experimental import pallas as pl
from jax.experimental.pallas import tpu as pltpu

DIM = 768
HIDDEN = 4 * DIM
NUM_EXPERTS = 8
TOPK = 8
B_TOTAL = 4096
SEL = NUM_EXPERTS * TOPK

HID_BLOCK = 512
N_HID_BLOCKS = HIDDEN // HID_BLOCK
FFN_STEPS = NUM_EXPERTS * N_HID_BLOCKS
TOK_BLOCK = 512
N_TOK_BLOCKS = B_TOTAL // TOK_BLOCK
N_STEPS = FFN_STEPS + N_TOK_BLOCKS

_INTERPRET = False


def _body(x_ref, wr_ref, br_ref, w1_ref, b1_ref, w2_ref, b2_ref, out_ref,
          oh01_ref, ohs_ref, y_ref):
    s = pl.program_id(0)

    @pl.when(s == 0)
    def _route():
        logits = jax.lax.dot_general(
            wr_ref[...], x_ref[...], (((1,), (1,)), ((), ())),
            preferred_element_type=jnp.float32,
            precision=jax.lax.Precision.HIGHEST)
        logits = logits + br_ref[...]
        mx = jnp.max(logits, axis=0, keepdims=True)
        ex = jnp.exp(logits - mx)
        sv = ex / jnp.sum(ex, axis=0, keepdims=True)

        lane = jax.lax.broadcasted_iota(jnp.int32, (NUM_EXPERTS, B_TOTAL), 1)
        krow = jax.lax.broadcasted_iota(jnp.int32, (SEL, B_TOTAL), 0) % TOPK
        oh01 = jnp.zeros((SEL, B_TOTAL), jnp.float32)
        ohs = jnp.zeros((SEL, B_TOTAL), jnp.float32)
        for k in range(TOPK):
            m = jnp.max(sv, axis=1, keepdims=True)
            amin = jnp.min(jnp.where(sv == m, lane, B_TOTAL), axis=1,
                           keepdims=True)
            sel = (lane == amin).astype(jnp.float32)
            sel_x = jnp.broadcast_to(sel[:, None, :],
                                     (NUM_EXPERTS, TOPK, B_TOTAL)
                                     ).reshape(SEL, B_TOTAL)
            m_x = jnp.broadcast_to(m[:, None, :],
                                   (NUM_EXPERTS, TOPK, 1)).reshape(SEL, 1)
            kmask = (krow == k).astype(jnp.float32)
            oh01 = oh01 + kmask * sel_x
            ohs = ohs + kmask * sel_x * m_x
            sv = jnp.where(lane == amin, -jnp.inf, sv)
        oh01_ref[...] = oh01
        ohs_ref[...] = ohs

    @pl.when(s < FFN_STEPS)
    def _ffn():
        e = s // N_HID_BLOCKS
        hb = s % N_HID_BLOCKS
        row = pl.multiple_of(e * TOPK, TOPK)
        @pl.when(hb == 0)
        def _():
            y_ref[pl.ds(row, TOPK), :] = jnp.zeros((TOPK, DIM), jnp.float32)
        xg = jax.lax.dot_general(
            oh01_ref[pl.ds(row, TOPK), :], x_ref[...],
            (((1,), (0,)), ((), ())), preferred_element_type=jnp.float32,
            precision=jax.lax.Precision.HIGHEST)
        h = jax.lax.dot_general(
            xg, w1_ref[0], (((1,), (1,)), ((), ())),
            preferred_element_type=jnp.float32)
        h = jnp.maximum(h + b1_ref[0], 0.0)
        y_ref[pl.ds(row, TOPK), :] += jax.lax.dot_general(
            h, w2_ref[0], (((1,), (1,)), ((), ())),
            preferred_element_type=jnp.float32)

        @pl.when(hb == N_HID_BLOCKS - 1)
        def _():
            y_ref[pl.ds(row, TOPK), :] += jnp.broadcast_to(
                b2_ref[0], (TOPK, DIM))

    @pl.when(s >= FFN_STEPS)
    def _out():
        t = s - FFN_STEPS
        oh_blk = ohs_ref[:, pl.ds(t * TOK_BLOCK, TOK_BLOCK)]
        out_ref[...] = jax.lax.dot_general(
            oh_blk, y_ref[...], (((0,), (0,)), ((), ())),
            preferred_element_type=jnp.float32,
            precision=jax.lax.Precision.HIGHEST)


def kernel(x, Wr, br, W1, b1, W2, b2):
    B, D = x.shape
    E = Wr.shape[0]

    def wmap(s):
        fs = jnp.minimum(s, FFN_STEPS - 1)
        return fs // N_HID_BLOCKS, fs % N_HID_BLOCKS

    out = pl.pallas_call(
        _body,
        grid=(N_STEPS,),
        in_specs=[
            pl.BlockSpec((B, D), lambda s: (0, 0)),
            pl.BlockSpec((E, D), lambda s: (0, 0)),
            pl.BlockSpec((E, 1), lambda s: (0, 0)),
            pl.BlockSpec((1, HID_BLOCK, D), lambda s: (wmap(s)[0], wmap(s)[1], 0)),
            pl.BlockSpec((1, 1, HID_BLOCK), lambda s: (wmap(s)[0], 0, wmap(s)[1])),
            pl.BlockSpec((1, D, HID_BLOCK), lambda s: (wmap(s)[0], 0, wmap(s)[1])),
            pl.BlockSpec((1, 1, D), lambda s: (wmap(s)[0], 0, 0)),
        ],
        out_specs=pl.BlockSpec(
            (TOK_BLOCK, D),
            lambda s: (jnp.maximum(s - FFN_STEPS, 0), 0)),
        out_shape=jax.ShapeDtypeStruct((B, D), jnp.float32),
        scratch_shapes=[
            pltpu.VMEM((SEL, B_TOTAL), jnp.float32),
            pltpu.VMEM((SEL, B_TOTAL), jnp.float32),
            pltpu.VMEM((SEL, DIM), jnp.float32),
        ],
        interpret=_INTERPRET,
    )(x, Wr, br.reshape(E, 1), W1, b1.reshape(E, 1, HIDDEN), W2,
      b2.reshape(E, 1, D))
    return out


# speedup vs baseline: 3.1962x; 3.1962x over previous
import jax
import jax.numpy as jnp
from jax.experimental import pallas as pl
from jax.experimental.pallas import tpu as pltpu

DIM = 768
HIDDEN = 4 * DIM
NUM_EXPERTS = 8
TOPK = 8
B_TOTAL = 4096
SEL = NUM_EXPERTS * TOPK

HID_BLOCK = 512
N_HID_BLOCKS = HIDDEN // HID_BLOCK
FFN_STEPS = NUM_EXPERTS * N_HID_BLOCKS
TOK_BLOCK = 512
N_TOK_BLOCKS = B_TOTAL // TOK_BLOCK
N_STEPS = FFN_STEPS + N_TOK_BLOCKS

_INTERPRET = False


def _body(x_ref, wr_ref, br_ref, w1_ref, b1_ref, w2_ref, b2_ref, out_ref,
          oh01_ref, ohs_ref, y_ref, xg_ref):
    s = pl.program_id(0)

    @pl.when(s == 0)
    def _route():
        logits = jax.lax.dot_general(
            wr_ref[...], x_ref[...], (((1,), (1,)), ((), ())),
            preferred_element_type=jnp.float32,
            precision=jax.lax.Precision.HIGHEST)
        logits = logits + br_ref[...]
        mx = jnp.max(logits, axis=0, keepdims=True)
        ex = jnp.exp(logits - mx)
        sv = ex / jnp.sum(ex, axis=0, keepdims=True)

        lane = jax.lax.broadcasted_iota(jnp.int32, (NUM_EXPERTS, B_TOTAL), 1)
        krow = jax.lax.broadcasted_iota(jnp.int32, (SEL, B_TOTAL), 0) % TOPK
        oh01 = jnp.zeros((SEL, B_TOTAL), jnp.float32)
        ohs = jnp.zeros((SEL, B_TOTAL), jnp.float32)
        for k in range(TOPK):
            m = jnp.max(sv, axis=1, keepdims=True)
            amin = jnp.min(jnp.where(sv == m, lane, B_TOTAL), axis=1,
                           keepdims=True)
            sel = (lane == amin).astype(jnp.float32)
            sel_x = jnp.broadcast_to(sel[:, None, :],
                                     (NUM_EXPERTS, TOPK, B_TOTAL)
                                     ).reshape(SEL, B_TOTAL)
            m_x = jnp.broadcast_to(m[:, None, :],
                                   (NUM_EXPERTS, TOPK, 1)).reshape(SEL, 1)
            kmask = (krow == k).astype(jnp.float32)
            oh01 = oh01 + kmask * sel_x
            ohs = ohs + kmask * sel_x * m_x
            sv = jnp.where(lane == amin, -jnp.inf, sv)
        oh01_ref[...] = oh01
        ohs_ref[...] = ohs
        xg_ref[...] = jax.lax.dot_general(
            oh01, x_ref[...], (((1,), (0,)), ((), ())),
            preferred_element_type=jnp.float32,
            precision=jax.lax.Precision.HIGHEST)

    @pl.when(s < FFN_STEPS)
    def _ffn():
        e = s // N_HID_BLOCKS
        hb = s % N_HID_BLOCKS
        row = pl.multiple_of(e * TOPK, TOPK)
        @pl.when(hb == 0)
        def _():
            y_ref[pl.ds(row, TOPK), :] = jnp.zeros((TOPK, DIM), jnp.float32)
        xg = xg_ref[pl.ds(row, TOPK), :]
        h = jax.lax.dot_general(
            xg, w1_ref[0], (((1,), (1,)), ((), ())),
            preferred_element_type=jnp.float32)
        h = jnp.maximum(h + b1_ref[0], 0.0)
        y_ref[pl.ds(row, TOPK), :] += jax.lax.dot_general(
            h, w2_ref[0], (((1,), (1,)), ((), ())),
            preferred_element_type=jnp.float32)

        @pl.when(hb == N_HID_BLOCKS - 1)
        def _():
            y_ref[pl.ds(row, TOPK), :] += jnp.broadcast_to(
                b2_ref[0], (TOPK, DIM))

    @pl.when(s >= FFN_STEPS)
    def _out():
        t = s - FFN_STEPS
        oh_blk = ohs_ref[:, pl.ds(t * TOK_BLOCK, TOK_BLOCK)]
        out_ref[...] = jax.lax.dot_general(
            oh_blk, y_ref[...], (((0,), (0,)), ((), ())),
            preferred_element_type=jnp.float32,
            precision=jax.lax.Precision.HIGHEST)


def kernel(x, Wr, br, W1, b1, W2, b2):
    B, D = x.shape
    E = Wr.shape[0]

    def wmap(s):
        fs = jnp.minimum(s, FFN_STEPS - 1)
        return fs // N_HID_BLOCKS, fs % N_HID_BLOCKS

    out = pl.pallas_call(
        _body,
        grid=(N_STEPS,),
        in_specs=[
            pl.BlockSpec((B, D), lambda s: (0, 0)),
            pl.BlockSpec((E, D), lambda s: (0, 0)),
            pl.BlockSpec((E, 1), lambda s: (0, 0)),
            pl.BlockSpec((1, HID_BLOCK, D), lambda s: (wmap(s)[0], wmap(s)[1], 0)),
            pl.BlockSpec((1, 1, HID_BLOCK), lambda s: (wmap(s)[0], 0, wmap(s)[1])),
            pl.BlockSpec((1, D, HID_BLOCK), lambda s: (wmap(s)[0], 0, wmap(s)[1])),
            pl.BlockSpec((1, 1, D), lambda s: (wmap(s)[0], 0, 0)),
        ],
        out_specs=pl.BlockSpec(
            (TOK_BLOCK, D),
            lambda s: (jnp.maximum(s - FFN_STEPS, 0), 0)),
        out_shape=jax.ShapeDtypeStruct((B, D), jnp.float32),
        scratch_shapes=[
            pltpu.VMEM((SEL, B_TOTAL), jnp.float32),
            pltpu.VMEM((SEL, B_TOTAL), jnp.float32),
            pltpu.VMEM((SEL, DIM), jnp.float32),
            pltpu.VMEM((SEL, DIM), jnp.float32),
        ],
        interpret=_INTERPRET,
    )(x, Wr, br.reshape(E, 1), W1, b1.reshape(E, 1, HIDDEN), W2,
      b2.reshape(E, 1, D))
    return out


# speedup vs baseline: 3.4563x; 1.0814x over previous
import jax
import jax.numpy as jnp
from jax.experimental import pallas as pl
from jax.experimental.pallas import tpu as pltpu

DIM = 768
HIDDEN = 4 * DIM
NUM_EXPERTS = 8
TOPK = 8
B_TOTAL = 4096
SEL = NUM_EXPERTS * TOPK

HID_BLOCK = 512
N_HID_BLOCKS = HIDDEN // HID_BLOCK
FFN_STEPS = NUM_EXPERTS * N_HID_BLOCKS
TOK_BLOCK = 512
N_TOK_BLOCKS = B_TOTAL // TOK_BLOCK
N_STEPS = FFN_STEPS + N_TOK_BLOCKS

_INTERPRET = False


def _body(x_ref, wr_ref, br_ref, w1_ref, b1_ref, w2_ref, b2_ref, out_ref,
          oh01_ref, ohs_ref, y_ref, xg_ref):
    s = pl.program_id(0)

    @pl.when(s == 0)
    def _route():
        logits = jax.lax.dot_general(
            wr_ref[...], x_ref[...], (((1,), (1,)), ((), ())),
            preferred_element_type=jnp.float32)
        logits = logits + br_ref[...]
        mx = jnp.max(logits, axis=0, keepdims=True)
        ex = jnp.exp(logits - mx)
        sv = ex / jnp.sum(ex, axis=0, keepdims=True)

        lane = jax.lax.broadcasted_iota(jnp.int32, (NUM_EXPERTS, B_TOTAL), 1)
        krow = jax.lax.broadcasted_iota(jnp.int32, (SEL, B_TOTAL), 0) % TOPK
        oh01 = jnp.zeros((SEL, B_TOTAL), jnp.float32)
        ohs = jnp.zeros((SEL, B_TOTAL), jnp.float32)
        for k in range(TOPK):
            m = jnp.max(sv, axis=1, keepdims=True)
            amin = jnp.min(jnp.where(sv == m, lane, B_TOTAL), axis=1,
                           keepdims=True)
            sel = (lane == amin).astype(jnp.float32)
            sel_x = jnp.broadcast_to(sel[:, None, :],
                                     (NUM_EXPERTS, TOPK, B_TOTAL)
                                     ).reshape(SEL, B_TOTAL)
            m_x = jnp.broadcast_to(m[:, None, :],
                                   (NUM_EXPERTS, TOPK, 1)).reshape(SEL, 1)
            kmask = (krow == k).astype(jnp.float32)
            oh01 = oh01 + kmask * sel_x
            ohs = ohs + kmask * sel_x * m_x
            sv = jnp.where(lane == amin, -jnp.inf, sv)
        oh01_ref[...] = oh01
        ohs_ref[...] = ohs
        xg_ref[...] = jax.lax.dot_general(
            oh01, x_ref[...], (((1,), (0,)), ((), ())),
            preferred_element_type=jnp.float32,
            precision=jax.lax.Precision.HIGHEST)

    @pl.when(s < FFN_STEPS)
    def _ffn():
        e = s // N_HID_BLOCKS
        hb = s % N_HID_BLOCKS
        row = pl.multiple_of(e * TOPK, TOPK)
        @pl.when(hb == 0)
        def _():
            y_ref[pl.ds(row, TOPK), :] = jnp.zeros((TOPK, DIM), jnp.float32)
        xg = xg_ref[pl.ds(row, TOPK), :]
        h = jax.lax.dot_general(
            xg, w1_ref[0], (((1,), (1,)), ((), ())),
            preferred_element_type=jnp.float32)
        h = jnp.maximum(h + b1_ref[0], 0.0)
        y_ref[pl.ds(row, TOPK), :] += jax.lax.dot_general(
            h, w2_ref[0], (((1,), (1,)), ((), ())),
            preferred_element_type=jnp.float32)

        @pl.when(hb == N_HID_BLOCKS - 1)
        def _():
            y_ref[pl.ds(row, TOPK), :] += jnp.broadcast_to(
                b2_ref[0], (TOPK, DIM))

    @pl.when(s >= FFN_STEPS)
    def _out():
        t = s - FFN_STEPS
        oh_blk = ohs_ref[:, pl.ds(t * TOK_BLOCK, TOK_BLOCK)]
        out_ref[...] = jax.lax.dot_general(
            oh_blk, y_ref[...], (((0,), (0,)), ((), ())),
            preferred_element_type=jnp.float32,
            precision=jax.lax.Precision.HIGHEST)


def kernel(x, Wr, br, W1, b1, W2, b2):
    B, D = x.shape
    E = Wr.shape[0]

    def wmap(s):
        fs = jnp.minimum(s, FFN_STEPS - 1)
        return fs // N_HID_BLOCKS, fs % N_HID_BLOCKS

    out = pl.pallas_call(
        _body,
        grid=(N_STEPS,),
        in_specs=[
            pl.BlockSpec((B, D), lambda s: (0, 0)),
            pl.BlockSpec((E, D), lambda s: (0, 0)),
            pl.BlockSpec((E, 1), lambda s: (0, 0)),
            pl.BlockSpec((1, HID_BLOCK, D), lambda s: (wmap(s)[0], wmap(s)[1], 0)),
            pl.BlockSpec((1, 1, HID_BLOCK), lambda s: (wmap(s)[0], 0, wmap(s)[1])),
            pl.BlockSpec((1, D, HID_BLOCK), lambda s: (wmap(s)[0], 0, wmap(s)[1])),
            pl.BlockSpec((1, 1, D), lambda s: (wmap(s)[0], 0, 0)),
        ],
        out_specs=pl.BlockSpec(
            (TOK_BLOCK, D),
            lambda s: (jnp.maximum(s - FFN_STEPS, 0), 0)),
        out_shape=jax.ShapeDtypeStruct((B, D), jnp.float32),
        scratch_shapes=[
            pltpu.VMEM((SEL, B_TOTAL), jnp.float32),
            pltpu.VMEM((SEL, B_TOTAL), jnp.float32),
            pltpu.VMEM((SEL, DIM), jnp.float32),
            pltpu.VMEM((SEL, DIM), jnp.float32),
        ],
        interpret=_INTERPRET,
    )(x, Wr, br.reshape(E, 1), W1, b1.reshape(E, 1, HIDDEN), W2,
      b2.reshape(E, 1, D))
    return out


# speedup vs baseline: 4.0394x; 1.1687x over previous
import jax
import jax.numpy as jnp
from jax.experimental import pallas as pl
from jax.experimental.pallas import tpu as pltpu

DIM = 768
HIDDEN = 4 * DIM
NUM_EXPERTS = 8
TOPK = 8
B_TOTAL = 4096
SEL = NUM_EXPERTS * TOPK

HID_BLOCK = 512
N_HID_BLOCKS = HIDDEN // HID_BLOCK
FFN_STEPS = NUM_EXPERTS * N_HID_BLOCKS
TOK_BLOCK = 512
N_TOK_BLOCKS = B_TOTAL // TOK_BLOCK
N_STEPS = FFN_STEPS + N_TOK_BLOCKS

_INTERPRET = False


def _body(x_ref, wr_ref, br_ref, w1_ref, b1_ref, w2_ref, b2_ref, out_ref,
          oh01_ref, ohs_ref, y_ref, xg_ref):
    s = pl.program_id(0)

    @pl.when(s == 0)
    def _route():
        logits = jax.lax.dot_general(
            wr_ref[...], x_ref[...], (((1,), (1,)), ((), ())),
            preferred_element_type=jnp.float32)
        logits = logits + br_ref[...]
        mx = jnp.max(logits, axis=0, keepdims=True)
        ex = jnp.exp(logits - mx)
        sv = ex / jnp.sum(ex, axis=0, keepdims=True)

        lane = jax.lax.broadcasted_iota(jnp.int32, (NUM_EXPERTS, B_TOTAL), 1)
        krow = jax.lax.broadcasted_iota(jnp.int32, (SEL, B_TOTAL), 0) % TOPK
        oh01 = jnp.zeros((SEL, B_TOTAL), jnp.float32)
        ohs = jnp.zeros((SEL, B_TOTAL), jnp.float32)
        for k in range(TOPK):
            m = jnp.max(sv, axis=1, keepdims=True)
            amin = jnp.min(jnp.where(sv == m, lane, B_TOTAL), axis=1,
                           keepdims=True)
            sel = (lane == amin).astype(jnp.float32)
            sel_x = jnp.broadcast_to(sel[:, None, :],
                                     (NUM_EXPERTS, TOPK, B_TOTAL)
                                     ).reshape(SEL, B_TOTAL)
            m_x = jnp.broadcast_to(m[:, None, :],
                                   (NUM_EXPERTS, TOPK, 1)).reshape(SEL, 1)
            kmask = (krow == k).astype(jnp.float32)
            oh01 = oh01 + kmask * sel_x
            ohs = ohs + kmask * sel_x * m_x
            sv = jnp.where(lane == amin, -jnp.inf, sv)
        oh01_ref[...] = oh01
        ohs_ref[...] = ohs
        xg_ref[...] = jax.lax.dot_general(
            oh01, x_ref[...], (((1,), (0,)), ((), ())),
            preferred_element_type=jnp.float32)

    @pl.when(s < FFN_STEPS)
    def _ffn():
        e = s // N_HID_BLOCKS
        hb = s % N_HID_BLOCKS
        row = pl.multiple_of(e * TOPK, TOPK)
        @pl.when(hb == 0)
        def _():
            y_ref[pl.ds(row, TOPK), :] = jnp.zeros((TOPK, DIM), jnp.float32)
        xg = xg_ref[pl.ds(row, TOPK), :]
        h = jax.lax.dot_general(
            xg, w1_ref[0], (((1,), (1,)), ((), ())),
            preferred_element_type=jnp.float32)
        h = jnp.maximum(h + b1_ref[0], 0.0)
        y_ref[pl.ds(row, TOPK), :] += jax.lax.dot_general(
            h, w2_ref[0], (((1,), (1,)), ((), ())),
            preferred_element_type=jnp.float32)

        @pl.when(hb == N_HID_BLOCKS - 1)
        def _():
            y_ref[pl.ds(row, TOPK), :] += jnp.broadcast_to(
                b2_ref[0], (TOPK, DIM))

    @pl.when(s >= FFN_STEPS)
    def _out():
        t = s - FFN_STEPS
        oh_blk = ohs_ref[:, pl.ds(t * TOK_BLOCK, TOK_BLOCK)]
        out_ref[...] = jax.lax.dot_general(
            oh_blk, y_ref[...], (((0,), (0,)), ((), ())),
            preferred_element_type=jnp.float32)


def kernel(x, Wr, br, W1, b1, W2, b2):
    B, D = x.shape
    E = Wr.shape[0]

    def wmap(s):
        fs = jnp.minimum(s, FFN_STEPS - 1)
        return fs // N_HID_BLOCKS, fs % N_HID_BLOCKS

    out = pl.pallas_call(
        _body,
        grid=(N_STEPS,),
        in_specs=[
            pl.BlockSpec((B, D), lambda s: (0, 0)),
            pl.BlockSpec((E, D), lambda s: (0, 0)),
            pl.BlockSpec((E, 1), lambda s: (0, 0)),
            pl.BlockSpec((1, HID_BLOCK, D), lambda s: (wmap(s)[0], wmap(s)[1], 0)),
            pl.BlockSpec((1, 1, HID_BLOCK), lambda s: (wmap(s)[0], 0, wmap(s)[1])),
            pl.BlockSpec((1, D, HID_BLOCK), lambda s: (wmap(s)[0], 0, wmap(s)[1])),
            pl.BlockSpec((1, 1, D), lambda s: (wmap(s)[0], 0, 0)),
        ],
        out_specs=pl.BlockSpec(
            (TOK_BLOCK, D),
            lambda s: (jnp.maximum(s - FFN_STEPS, 0), 0)),
        out_shape=jax.ShapeDtypeStruct((B, D), jnp.float32),
        scratch_shapes=[
            pltpu.VMEM((SEL, B_TOTAL), jnp.float32),
            pltpu.VMEM((SEL, B_TOTAL), jnp.float32),
            pltpu.VMEM((SEL, DIM), jnp.float32),
            pltpu.VMEM((SEL, DIM), jnp.float32),
        ],
        interpret=_INTERPRET,
    )(x, Wr, br.reshape(E, 1), W1, b1.reshape(E, 1, HIDDEN), W2,
      b2.reshape(E, 1, D))
    return out


# speedup vs baseline: 4.8829x; 1.2088x over previous
import jax
import jax.numpy as jnp
from jax.experimental import pallas as pl
from jax.experimental.pallas import tpu as pltpu

DIM = 768
HIDDEN = 4 * DIM
NUM_EXPERTS = 8
TOPK = 8
B_TOTAL = 4096
SEL = NUM_EXPERTS * TOPK

HID_BLOCK = 1024
N_HID_BLOCKS = HIDDEN // HID_BLOCK
FFN_STEPS = NUM_EXPERTS * N_HID_BLOCKS
TOK_BLOCK = 1024
N_TOK_BLOCKS = B_TOTAL // TOK_BLOCK
N_STEPS = FFN_STEPS + N_TOK_BLOCKS

_INTERPRET = False


def _body(x_ref, wr_ref, br_ref, w1_ref, b1_ref, w2_ref, b2_ref, out_ref,
          oh01_ref, ohs_ref, y_ref, xg_ref):
    s = pl.program_id(0)

    @pl.when(s == 0)
    def _route():
        logits = jax.lax.dot_general(
            wr_ref[...], x_ref[...], (((1,), (1,)), ((), ())),
            preferred_element_type=jnp.float32)
        logits = logits + br_ref[...]
        mx = jnp.max(logits, axis=0, keepdims=True)
        ex = jnp.exp(logits - mx)
        sv = ex / jnp.sum(ex, axis=0, keepdims=True)

        lane = jax.lax.broadcasted_iota(jnp.int32, (NUM_EXPERTS, B_TOTAL), 1)
        krow = jax.lax.broadcasted_iota(jnp.int32, (SEL, B_TOTAL), 0) % TOPK
        oh01 = jnp.zeros((SEL, B_TOTAL), jnp.float32)
        ohs = jnp.zeros((SEL, B_TOTAL), jnp.float32)
        for k in range(TOPK):
            m = jnp.max(sv, axis=1, keepdims=True)
            amin = jnp.min(jnp.where(sv == m, lane, B_TOTAL), axis=1,
                           keepdims=True)
            sel = (lane == amin).astype(jnp.float32)
            sel_x = jnp.broadcast_to(sel[:, None, :],
                                     (NUM_EXPERTS, TOPK, B_TOTAL)
                                     ).reshape(SEL, B_TOTAL)
            m_x = jnp.broadcast_to(m[:, None, :],
                                   (NUM_EXPERTS, TOPK, 1)).reshape(SEL, 1)
            kmask = (krow == k).astype(jnp.float32)
            oh01 = oh01 + kmask * sel_x
            ohs = ohs + kmask * sel_x * m_x
            sv = jnp.where(lane == amin, -jnp.inf, sv)
        oh01_ref[...] = oh01
        ohs_ref[...] = ohs
        xg_ref[...] = jax.lax.dot_general(
            oh01, x_ref[...], (((1,), (0,)), ((), ())),
            preferred_element_type=jnp.float32)

    @pl.when(s < FFN_STEPS)
    def _ffn():
        e = s // N_HID_BLOCKS
        hb = s % N_HID_BLOCKS
        row = pl.multiple_of(e * TOPK, TOPK)
        @pl.when(hb == 0)
        def _():
            y_ref[pl.ds(row, TOPK), :] = jnp.zeros((TOPK, DIM), jnp.float32)
        xg = xg_ref[pl.ds(row, TOPK), :]
        h = jax.lax.dot_general(
            xg, w1_ref[0], (((1,), (1,)), ((), ())),
            preferred_element_type=jnp.float32)
        h = jnp.maximum(h + b1_ref[0], 0.0)
        y_ref[pl.ds(row, TOPK), :] += jax.lax.dot_general(
            h, w2_ref[0], (((1,), (1,)), ((), ())),
            preferred_element_type=jnp.float32)

        @pl.when(hb == N_HID_BLOCKS - 1)
        def _():
            y_ref[pl.ds(row, TOPK), :] += jnp.broadcast_to(
                b2_ref[0], (TOPK, DIM))

    @pl.when(s >= FFN_STEPS)
    def _out():
        t = s - FFN_STEPS
        oh_blk = ohs_ref[:, pl.ds(t * TOK_BLOCK, TOK_BLOCK)]
        out_ref[...] = jax.lax.dot_general(
            oh_blk, y_ref[...], (((0,), (0,)), ((), ())),
            preferred_element_type=jnp.float32)


def kernel(x, Wr, br, W1, b1, W2, b2):
    B, D = x.shape
    E = Wr.shape[0]

    def wmap(s):
        fs = jnp.minimum(s, FFN_STEPS - 1)
        return fs // N_HID_BLOCKS, fs % N_HID_BLOCKS

    out = pl.pallas_call(
        _body,
        grid=(N_STEPS,),
        in_specs=[
            pl.BlockSpec((B, D), lambda s: (0, 0)),
            pl.BlockSpec((E, D), lambda s: (0, 0)),
            pl.BlockSpec((E, 1), lambda s: (0, 0)),
            pl.BlockSpec((1, HID_BLOCK, D), lambda s: (wmap(s)[0], wmap(s)[1], 0)),
            pl.BlockSpec((1, 1, HID_BLOCK), lambda s: (wmap(s)[0], 0, wmap(s)[1])),
            pl.BlockSpec((1, D, HID_BLOCK), lambda s: (wmap(s)[0], 0, wmap(s)[1])),
            pl.BlockSpec((1, 1, D), lambda s: (wmap(s)[0], 0, 0)),
        ],
        out_specs=pl.BlockSpec(
            (TOK_BLOCK, D),
            lambda s: (jnp.maximum(s - FFN_STEPS, 0), 0)),
        out_shape=jax.ShapeDtypeStruct((B, D), jnp.float32),
        scratch_shapes=[
            pltpu.VMEM((SEL, B_TOTAL), jnp.float32),
            pltpu.VMEM((SEL, B_TOTAL), jnp.float32),
            pltpu.VMEM((SEL, DIM), jnp.float32),
            pltpu.VMEM((SEL, DIM), jnp.float32),
        ],
        interpret=_INTERPRET,
    )(x, Wr, br.reshape(E, 1), W1, b1.reshape(E, 1, HIDDEN), W2,
      b2.reshape(E, 1, D))
    return out


# speedup vs baseline: 5.1274x; 1.0501x over previous
import jax
import jax.numpy as jnp
from jax.experimental import pallas as pl
from jax.experimental.pallas import tpu as pltpu

DIM = 768
HIDDEN = 4 * DIM
NUM_EXPERTS = 8
TOPK = 8
B_TOTAL = 4096
SEL = NUM_EXPERTS * TOPK

HID_BLOCK = 1536
N_HID_BLOCKS = HIDDEN // HID_BLOCK
FFN_STEPS = NUM_EXPERTS * N_HID_BLOCKS
TOK_BLOCK = 1024
N_TOK_BLOCKS = B_TOTAL // TOK_BLOCK
N_STEPS = FFN_STEPS + N_TOK_BLOCKS

_INTERPRET = False


def _body(x_ref, wr_ref, br_ref, w1_ref, b1_ref, w2_ref, b2_ref, out_ref,
          oh01_ref, ohs_ref, y_ref, xg_ref):
    s = pl.program_id(0)

    @pl.when(s == 0)
    def _route():
        logits = jax.lax.dot_general(
            wr_ref[...], x_ref[...], (((1,), (1,)), ((), ())),
            preferred_element_type=jnp.float32)
        logits = logits + br_ref[...]
        mx = jnp.max(logits, axis=0, keepdims=True)
        ex = jnp.exp(logits - mx)
        sv = ex / jnp.sum(ex, axis=0, keepdims=True)

        lane = jax.lax.broadcasted_iota(jnp.int32, (NUM_EXPERTS, B_TOTAL), 1)
        krow = jax.lax.broadcasted_iota(jnp.int32, (SEL, B_TOTAL), 0) % TOPK
        oh01 = jnp.zeros((SEL, B_TOTAL), jnp.float32)
        ohs = jnp.zeros((SEL, B_TOTAL), jnp.float32)
        for k in range(TOPK):
            m = jnp.max(sv, axis=1, keepdims=True)
            amin = jnp.min(jnp.where(sv == m, lane, B_TOTAL), axis=1,
                           keepdims=True)
            sel = (lane == amin).astype(jnp.float32)
            sel_x = jnp.broadcast_to(sel[:, None, :],
                                     (NUM_EXPERTS, TOPK, B_TOTAL)
                                     ).reshape(SEL, B_TOTAL)
            m_x = jnp.broadcast_to(m[:, None, :],
                                   (NUM_EXPERTS, TOPK, 1)).reshape(SEL, 1)
            kmask = (krow == k).astype(jnp.float32)
            oh01 = oh01 + kmask * sel_x
            ohs = ohs + kmask * sel_x * m_x
            sv = jnp.where(lane == amin, -jnp.inf, sv)
        oh01_ref[...] = oh01
        ohs_ref[...] = ohs
        xg_ref[...] = jax.lax.dot_general(
            oh01, x_ref[...], (((1,), (0,)), ((), ())),
            preferred_element_type=jnp.float32)

    @pl.when(s < FFN_STEPS)
    def _ffn():
        e = s // N_HID_BLOCKS
        hb = s % N_HID_BLOCKS
        row = pl.multiple_of(e * TOPK, TOPK)
        @pl.when(hb == 0)
        def _():
            y_ref[pl.ds(row, TOPK), :] = jnp.zeros((TOPK, DIM), jnp.float32)
        xg = xg_ref[pl.ds(row, TOPK), :]
        h = jax.lax.dot_general(
            xg, w1_ref[0], (((1,), (1,)), ((), ())),
            preferred_element_type=jnp.float32)
        h = jnp.maximum(h + b1_ref[0], 0.0)
        y_ref[pl.ds(row, TOPK), :] += jax.lax.dot_general(
            h, w2_ref[0], (((1,), (1,)), ((), ())),
            preferred_element_type=jnp.float32)

        @pl.when(hb == N_HID_BLOCKS - 1)
        def _():
            y_ref[pl.ds(row, TOPK), :] += jnp.broadcast_to(
                b2_ref[0], (TOPK, DIM))

    @pl.when(s >= FFN_STEPS)
    def _out():
        t = s - FFN_STEPS
        oh_blk = ohs_ref[:, pl.ds(t * TOK_BLOCK, TOK_BLOCK)]
        out_ref[...] = jax.lax.dot_general(
            oh_blk, y_ref[...], (((0,), (0,)), ((), ())),
            preferred_element_type=jnp.float32)


def kernel(x, Wr, br, W1, b1, W2, b2):
    B, D = x.shape
    E = Wr.shape[0]

    def wmap(s):
        fs = jnp.minimum(s, FFN_STEPS - 1)
        return fs // N_HID_BLOCKS, fs % N_HID_BLOCKS

    out = pl.pallas_call(
        _body,
        grid=(N_STEPS,),
        in_specs=[
            pl.BlockSpec((B, D), lambda s: (0, 0)),
            pl.BlockSpec((E, D), lambda s: (0, 0)),
            pl.BlockSpec((E, 1), lambda s: (0, 0)),
            pl.BlockSpec((1, HID_BLOCK, D), lambda s: (wmap(s)[0], wmap(s)[1], 0)),
            pl.BlockSpec((1, 1, HID_BLOCK), lambda s: (wmap(s)[0], 0, wmap(s)[1])),
            pl.BlockSpec((1, D, HID_BLOCK), lambda s: (wmap(s)[0], 0, wmap(s)[1])),
            pl.BlockSpec((1, 1, D), lambda s: (wmap(s)[0], 0, 0)),
        ],
        out_specs=pl.BlockSpec(
            (TOK_BLOCK, D),
            lambda s: (jnp.maximum(s - FFN_STEPS, 0), 0)),
        out_shape=jax.ShapeDtypeStruct((B, D), jnp.float32),
        scratch_shapes=[
            pltpu.VMEM((SEL, B_TOTAL), jnp.float32),
            pltpu.VMEM((SEL, B_TOTAL), jnp.float32),
            pltpu.VMEM((SEL, DIM), jnp.float32),
            pltpu.VMEM((SEL, DIM), jnp.float32),
        ],
        interpret=_INTERPRET,
    )(x, Wr, br.reshape(E, 1), W1, b1.reshape(E, 1, HIDDEN), W2,
      b2.reshape(E, 1, D))
    return out


# speedup vs baseline: 5.1671x; 1.0077x over previous
import jax
import jax.numpy as jnp
from jax.experimental import pallas as pl
from jax.experimental.pallas import tpu as pltpu

DIM = 768
HIDDEN = 4 * DIM
NUM_EXPERTS = 8
TOPK = 8
B_TOTAL = 4096
SEL = NUM_EXPERTS * TOPK

HID_BLOCK = 1536
N_HID_BLOCKS = HIDDEN // HID_BLOCK
FFN_STEPS = NUM_EXPERTS * N_HID_BLOCKS
TOK_BLOCK = 2048
N_TOK_BLOCKS = B_TOTAL // TOK_BLOCK
N_STEPS = FFN_STEPS + N_TOK_BLOCKS

_INTERPRET = False


def _body(x_ref, wr_ref, br_ref, w1_ref, b1_ref, w2_ref, b2_ref, out_ref,
          oh01_ref, sc_ref, y_ref, xg_ref):
    s = pl.program_id(0)

    @pl.when(s == 0)
    def _route():
        logits = jax.lax.dot_general(
            wr_ref[...], x_ref[...], (((1,), (1,)), ((), ())),
            preferred_element_type=jnp.float32)
        logits = logits + br_ref[...]
        mx = jnp.max(logits, axis=0, keepdims=True)
        ex = jnp.exp(logits - mx)
        sv = ex / jnp.sum(ex, axis=0, keepdims=True)

        lane = jax.lax.broadcasted_iota(jnp.int32, (NUM_EXPERTS, B_TOTAL), 1)
        krow = jax.lax.broadcasted_iota(jnp.int32, (SEL, B_TOTAL), 0) % TOPK
        krow1 = krow[:, :1]
        oh01 = jnp.zeros((SEL, B_TOTAL), jnp.float32)
        scv = jnp.zeros((SEL, 1), jnp.float32)
        for k in range(TOPK):
            m = jnp.max(sv, axis=1, keepdims=True)
            amin = jnp.min(jnp.where(sv == m, lane, B_TOTAL), axis=1,
                           keepdims=True)
            sel = (lane == amin).astype(jnp.float32)
            sel_x = jnp.broadcast_to(sel[:, None, :],
                                     (NUM_EXPERTS, TOPK, B_TOTAL)
                                     ).reshape(SEL, B_TOTAL)
            m_x = jnp.broadcast_to(m[:, None, :],
                                   (NUM_EXPERTS, TOPK, 1)).reshape(SEL, 1)
            kmask = (krow == k).astype(jnp.float32)
            oh01 = oh01 + kmask * sel_x
            scv = scv + (krow1 == k).astype(jnp.float32) * m_x
            sv = jnp.where(lane == amin, -jnp.inf, sv)
        oh01_ref[...] = oh01
        sc_ref[...] = scv
        xg_ref[...] = jax.lax.dot_general(
            oh01, x_ref[...], (((1,), (0,)), ((), ())),
            preferred_element_type=jnp.float32)

    @pl.when(s < FFN_STEPS)
    def _ffn():
        e = s // N_HID_BLOCKS
        hb = s % N_HID_BLOCKS
        row = pl.multiple_of(e * TOPK, TOPK)
        @pl.when(hb == 0)
        def _():
            y_ref[pl.ds(row, TOPK), :] = jnp.zeros((TOPK, DIM), jnp.float32)
        xg = xg_ref[pl.ds(row, TOPK), :]
        h = jax.lax.dot_general(
            xg, w1_ref[0], (((1,), (1,)), ((), ())),
            preferred_element_type=jnp.float32)
        h = jnp.maximum(h + b1_ref[0], 0.0)
        y_ref[pl.ds(row, TOPK), :] += jax.lax.dot_general(
            h, w2_ref[0], (((1,), (1,)), ((), ())),
            preferred_element_type=jnp.float32)

        @pl.when(hb == N_HID_BLOCKS - 1)
        def _():
            yf = y_ref[pl.ds(row, TOPK), :] + jnp.broadcast_to(
                b2_ref[0], (TOPK, DIM))
            y_ref[pl.ds(row, TOPK), :] = yf * sc_ref[pl.ds(row, TOPK), :]

    @pl.when(s >= FFN_STEPS)
    def _out():
        t = s - FFN_STEPS
        oh_blk = oh01_ref[:, pl.ds(t * TOK_BLOCK, TOK_BLOCK)]
        out_ref[...] = jax.lax.dot_general(
            oh_blk, y_ref[...], (((0,), (0,)), ((), ())),
            preferred_element_type=jnp.float32)


def kernel(x, Wr, br, W1, b1, W2, b2):
    B, D = x.shape
    E = Wr.shape[0]

    def wmap(s):
        fs = jnp.minimum(s, FFN_STEPS - 1)
        return fs // N_HID_BLOCKS, fs % N_HID_BLOCKS

    out = pl.pallas_call(
        _body,
        grid=(N_STEPS,),
        in_specs=[
            pl.BlockSpec((B, D), lambda s: (0, 0)),
            pl.BlockSpec((E, D), lambda s: (0, 0)),
            pl.BlockSpec((E, 1), lambda s: (0, 0)),
            pl.BlockSpec((1, HID_BLOCK, D), lambda s: (wmap(s)[0], wmap(s)[1], 0)),
            pl.BlockSpec((1, 1, HID_BLOCK), lambda s: (wmap(s)[0], 0, wmap(s)[1])),
            pl.BlockSpec((1, D, HID_BLOCK), lambda s: (wmap(s)[0], 0, wmap(s)[1])),
            pl.BlockSpec((1, 1, D), lambda s: (wmap(s)[0], 0, 0)),
        ],
        out_specs=pl.BlockSpec(
            (TOK_BLOCK, D),
            lambda s: (jnp.maximum(s - FFN_STEPS, 0), 0)),
        out_shape=jax.ShapeDtypeStruct((B, D), jnp.float32),
        scratch_shapes=[
            pltpu.VMEM((SEL, B_TOTAL), jnp.float32),
            pltpu.VMEM((SEL, 1), jnp.float32),
            pltpu.VMEM((SEL, DIM), jnp.float32),
            pltpu.VMEM((SEL, DIM), jnp.float32),
        ],
        interpret=_INTERPRET,
    )(x, Wr, br.reshape(E, 1), W1, b1.reshape(E, 1, HIDDEN), W2,
      b2.reshape(E, 1, D))
    return out
